# baseline (speedup 1.0000x reference)
import functools

import jax
import jax.numpy as jnp
from jax import lax
from jax.experimental import pallas as pl
from jax.experimental.pallas import tpu as pltpu
from jax.experimental.pallas import tpu_sc as plsc

Q = 1024
DIM = 64
KDIM = 65536
TOPK = 32
LANES = 128
FR = 32
TK = 512
G = TK // FR
NT = KDIM // TK

NEG = float("-inf")


def _lane_top2(simT):
    best = simT[0:FR, :]
    bestg = jnp.zeros((FR, Q), jnp.int32)
    second = jnp.full((FR, Q), NEG)
    secondg = jnp.zeros((FR, Q), jnp.int32)
    for g in range(1, G):
        v = simT[g * FR:(g + 1) * FR, :]
        gt_best = v > best
        gt_second = v > second
        second = jnp.where(gt_best, best, jnp.where(gt_second, v, second))
        secondg = jnp.where(gt_best, bestg,
                            jnp.where(gt_second, g, secondg))
        best = jnp.where(gt_best, v, best)
        bestg = jnp.where(gt_best, g, bestg)
    return best, bestg, second, secondg


def _recompute_second(simT, m):
    second = jnp.full((FR, Q), NEG)
    secondg = jnp.zeros((FR, Q), jnp.int32)
    for g in range(G):
        v = simT[g * FR:(g + 1) * FR, :]
        gt = (v < m) & (v > second)
        secondg = jnp.where(gt, g, secondg)
        second = jnp.where(gt, v, second)
    return second, secondg


def _topk_body(q_ref, x_ref, w_ref, i_ref, p_ref, v_scr, i_scr):
    t = pl.program_id(0)

    @pl.when(t == 0)
    def _():
        v_scr[...] = jnp.full((TOPK, Q), NEG)
        i_scr[...] = jnp.zeros((TOPK, Q), jnp.int32)

    simT = jnp.dot(q_ref[...], x_ref[...], preferred_element_type=jnp.float32)
    m, a, m2, a2 = _lane_top2(simT)

    slot_iota = lax.broadcasted_iota(jnp.int32, (FR, Q), 0)
    is_row0 = lax.broadcasted_iota(jnp.int32, (TOPK, Q), 0) == 0

    def argmax0(arr, mx):
        return jnp.min(jnp.where(arr == mx, slot_iota, FR),
                       axis=0, keepdims=True)

    rv0, ri0 = v_scr[...], i_scr[...]
    mv0 = jnp.max(m, axis=0, keepdims=True)
    lstar0 = argmax0(m, mv0)
    go0 = jnp.any(mv0 > rv0[TOPK - 1:, :])

    def cond(carry):
        return carry[9]

    def body(carry):
        m, a, m2, a2, stale, rv, ri, lstar, mv, go = carry
        active = mv > rv[TOPK - 1:, :]
        ohl = slot_iota == lstar

        stale_at = jnp.max(jnp.where(ohl, stale, 0), axis=0, keepdims=True)
        need = jnp.any((stale_at > 0) & active)
        m2, a2 = lax.cond(need, lambda: _recompute_second(simT, m),
                          lambda: (m2, a2))
        stale = jnp.where(need, jnp.zeros((FR, Q), jnp.int32), stale)

        gstar = jnp.max(jnp.where(ohl, a, 0), axis=0, keepdims=True)
        idx = t * TK + gstar * FR + lstar

        v = jnp.where(active, mv, NEG)
        ge = rv >= v
        sh_v = jnp.concatenate([rv[:1, :], rv[:TOPK - 1, :]], axis=0)
        sh_i = jnp.concatenate([ri[:1, :], ri[:TOPK - 1, :]], axis=0)
        gep = (sh_v >= v) | is_row0
        rv = jnp.where(ge, rv, jnp.where(gep, v, sh_v))
        ri = jnp.where(ge, ri, jnp.where(gep, idx, sh_i))

        onehot = ohl & active
        m = jnp.where(onehot, m2, m)
        a = jnp.where(onehot, a2, a)
        stale = stale + onehot.astype(jnp.int32)

        mv = jnp.max(m, axis=0, keepdims=True)
        lstar = argmax0(m, mv)
        go = jnp.any(mv > rv[TOPK - 1:, :])
        return m, a, m2, a2, stale, rv, ri, lstar, mv, go

    stale0 = jnp.zeros((FR, Q), jnp.int32)
    out = lax.while_loop(cond, body,
                         (m, a, m2, a2, stale0, rv0, ri0, lstar0, mv0, go0))
    v_scr[...] = out[5]
    i_scr[...] = out[6]

    @pl.when(t == NT - 1)
    def _():
        rv = out[5]
        e = jnp.exp(rv - rv[:1, :])
        w_ref[...] = e / jnp.sum(e, axis=0, keepdims=True)
        i_ref[...] = out[6]
        p_ref[...] = lax.shift_right_logical(out[6], 1)


def _run_topk(qt, xT):
    return pl.pallas_call(
        _topk_body,
        grid=(NT,),
        in_specs=[
            pl.BlockSpec((TK, DIM), lambda t: (t, 0)),
            pl.BlockSpec((DIM, Q), lambda t: (0, 0)),
        ],
        out_specs=[
            pl.BlockSpec((TOPK, Q), lambda t: (0, 0)),
            pl.BlockSpec((TOPK, Q), lambda t: (0, 0)),
            pl.BlockSpec((TOPK, Q), lambda t: (0, 0)),
        ],
        out_shape=[
            jax.ShapeDtypeStruct((TOPK, Q), jnp.float32),
            jax.ShapeDtypeStruct((TOPK, Q), jnp.int32),
            jax.ShapeDtypeStruct((TOPK, Q), jnp.int32),
        ],
        scratch_shapes=[
            pltpu.VMEM((TOPK, Q), jnp.float32),
            pltpu.VMEM((TOPK, Q), jnp.int32),
        ],
    )(qt, xT)


@functools.cache
def _make_gather():
    info = plsc.get_sparse_core_info()
    nw = info.num_cores * info.num_subcores
    b = Q * TOPK
    b_per_w = b // nw
    n_chunk = b_per_w // LANES
    n_buf = 4
    mesh = plsc.VectorSubcoreMesh(core_axis_name="c", subcore_axis_name="s")

    @functools.partial(
        pl.kernel, mesh=mesh,
        out_type=jax.ShapeDtypeStruct((b, 2 * DIM), jnp.float32),
        scratch_types=[
            pltpu.VMEM((n_chunk, LANES), jnp.int32),
            pltpu.VMEM((n_buf * LANES, 2 * DIM), jnp.float32),
            pltpu.SemaphoreType.DMA,
        ],
    )
    def gather(table_hbm, pidx_hbm, out_hbm, idx_v, rows_v, sem):
        wid = lax.axis_index("s") * info.num_cores + lax.axis_index("c")
        base = wid * b_per_w
        pltpu.sync_copy(pidx_hbm.at[wid], idx_v)
        for wave in range(n_chunk // n_buf):
            for j in range(n_buf):
                pltpu.async_copy(
                    table_hbm.at[idx_v.at[wave * n_buf + j]],
                    rows_v.at[pl.ds(j * LANES, LANES)], sem)
            for j in range(n_buf):
                pltpu.make_async_copy(
                    table_hbm.at[idx_v.at[wave * n_buf + j]],
                    rows_v.at[pl.ds(j * LANES, LANES)], sem).wait()
            pltpu.sync_copy(
                rows_v,
                out_hbm.at[pl.ds(base + wave * n_buf * LANES, n_buf * LANES)])

    return gather


def _combine_body(w_ref, i_ref, g_ref, out_ref):
    par = (i_ref[...] & 1)[:, :, None] == 1
    rows = jnp.where(par, g_ref[:, :, DIM:], g_ref[:, :, :DIM])
    out_ref[...] = jnp.sum(w_ref[...][:, :, None] * rows, axis=1)


def _run_combine(w, idx, g):
    return pl.pallas_call(
        _combine_body,
        out_shape=jax.ShapeDtypeStruct((Q, DIM), jnp.float32),
    )(w, idx, g)


def kernel(x, queue):
    x2 = x.reshape(Q, DIM)
    qt = queue.T
    w_t, idx_t, pidx_t = _run_topk(qt, x2.T)
    w, idx, pidx = w_t.T, idx_t.T, pidx_t.T
    table = qt.reshape(KDIM // 2, 2 * DIM)
    g = _make_gather()(table, pidx.reshape(32, TOPK * Q // 32 // LANES, LANES))
    out = _run_combine(w, idx, g.reshape(Q, TOPK, 2 * DIM))
    return out.reshape(8, 128, DIM), idx.reshape(8, 128, TOPK)


# speedup vs baseline: 1.2311x; 1.2311x over previous
import functools

import jax
import jax.numpy as jnp
from jax import lax
from jax.experimental import pallas as pl
from jax.experimental.pallas import tpu as pltpu
from jax.experimental.pallas import tpu_sc as plsc

Q = 1024
DIM = 64
KDIM = 65536
TOPK = 32
LANES = 128
FR = 16
TK = 1024
G = TK // FR
NT = KDIM // TK

NEG = float("-inf")


def _lane_top2(simT):
    best = simT[0:FR, :]
    bestg = jnp.zeros((FR, Q), jnp.int32)
    second = jnp.full((FR, Q), NEG)
    secondg = jnp.zeros((FR, Q), jnp.int32)
    for g in range(1, G):
        v = simT[g * FR:(g + 1) * FR, :]
        gt_best = v > best
        gt_second = v > second
        second = jnp.where(gt_best, best, jnp.where(gt_second, v, second))
        secondg = jnp.where(gt_best, bestg,
                            jnp.where(gt_second, g, secondg))
        best = jnp.where(gt_best, v, best)
        bestg = jnp.where(gt_best, g, bestg)
    return best, bestg, second, secondg


def _recompute_second(simT, m):
    second = jnp.full((FR, Q), NEG)
    secondg = jnp.zeros((FR, Q), jnp.int32)
    for g in range(G):
        v = simT[g * FR:(g + 1) * FR, :]
        gt = (v < m) & (v > second)
        secondg = jnp.where(gt, g, secondg)
        second = jnp.where(gt, v, second)
    return second, secondg


def _topk_body(q_ref, x_ref, w_ref, i_ref, p_ref, v_scr, i_scr):
    t = pl.program_id(0)

    @pl.when(t == 0)
    def _():
        v_scr[...] = jnp.full((TOPK, Q), NEG)
        i_scr[...] = jnp.zeros((TOPK, Q), jnp.int32)

    simT = jnp.dot(q_ref[...], x_ref[...], preferred_element_type=jnp.float32)
    m, a, m2, a2 = _lane_top2(simT)

    slot_iota = lax.broadcasted_iota(jnp.int32, (FR, Q), 0)
    is_row0 = lax.broadcasted_iota(jnp.int32, (TOPK, Q), 0) == 0

    def argmax0(arr, mx):
        return jnp.min(jnp.where(arr == mx, slot_iota, FR),
                       axis=0, keepdims=True)

    rv0, ri0 = v_scr[...], i_scr[...]
    mv0 = jnp.max(m, axis=0, keepdims=True)
    lstar0 = argmax0(m, mv0)
    go0 = jnp.any(mv0 > rv0[TOPK - 1:, :])

    def cond(carry):
        return carry[9]

    def body(carry):
        m, a, m2, a2, stale, rv, ri, lstar, mv, go = carry
        active = mv > rv[TOPK - 1:, :]
        ohl = slot_iota == lstar

        stale_at = jnp.max(jnp.where(ohl, stale, 0), axis=0, keepdims=True)
        need = jnp.any((stale_at > 0) & active)
        m2, a2 = lax.cond(need, lambda: _recompute_second(simT, m),
                          lambda: (m2, a2))
        stale = jnp.where(need, jnp.zeros((FR, Q), jnp.int32), stale)

        gstar = jnp.max(jnp.where(ohl, a, 0), axis=0, keepdims=True)
        idx = t * TK + gstar * FR + lstar

        v = jnp.where(active, mv, NEG)
        ge = rv >= v
        sh_v = jnp.concatenate([rv[:1, :], rv[:TOPK - 1, :]], axis=0)
        sh_i = jnp.concatenate([ri[:1, :], ri[:TOPK - 1, :]], axis=0)
        gep = (sh_v >= v) | is_row0
        rv = jnp.where(ge, rv, jnp.where(gep, v, sh_v))
        ri = jnp.where(ge, ri, jnp.where(gep, idx, sh_i))

        onehot = ohl & active
        m = jnp.where(onehot, m2, m)
        a = jnp.where(onehot, a2, a)
        stale = stale + onehot.astype(jnp.int32)

        mv = jnp.max(m, axis=0, keepdims=True)
        lstar = argmax0(m, mv)
        go = jnp.any(mv > rv[TOPK - 1:, :])
        return m, a, m2, a2, stale, rv, ri, lstar, mv, go

    stale0 = jnp.zeros((FR, Q), jnp.int32)
    out = lax.while_loop(cond, body,
                         (m, a, m2, a2, stale0, rv0, ri0, lstar0, mv0, go0))
    v_scr[...] = out[5]
    i_scr[...] = out[6]

    @pl.when(t == NT - 1)
    def _():
        rv = out[5]
        e = jnp.exp(rv - rv[:1, :])
        w_ref[...] = e / jnp.sum(e, axis=0, keepdims=True)
        i_ref[...] = out[6]
        p_ref[...] = lax.shift_right_logical(out[6], 1)


def _run_topk(qt, xT):
    return pl.pallas_call(
        _topk_body,
        grid=(NT,),
        in_specs=[
            pl.BlockSpec((TK, DIM), lambda t: (t, 0)),
            pl.BlockSpec((DIM, Q), lambda t: (0, 0)),
        ],
        out_specs=[
            pl.BlockSpec((TOPK, Q), lambda t: (0, 0)),
            pl.BlockSpec((TOPK, Q), lambda t: (0, 0)),
            pl.BlockSpec((TOPK, Q), lambda t: (0, 0)),
        ],
        out_shape=[
            jax.ShapeDtypeStruct((TOPK, Q), jnp.float32),
            jax.ShapeDtypeStruct((TOPK, Q), jnp.int32),
            jax.ShapeDtypeStruct((TOPK, Q), jnp.int32),
        ],
        scratch_shapes=[
            pltpu.VMEM((TOPK, Q), jnp.float32),
            pltpu.VMEM((TOPK, Q), jnp.int32),
        ],
    )(qt, xT)


@functools.cache
def _make_gather():
    info = plsc.get_sparse_core_info()
    nw = info.num_cores * info.num_subcores
    b = Q * TOPK
    b_per_w = b // nw
    n_chunk = b_per_w // LANES
    n_buf = 4
    mesh = plsc.VectorSubcoreMesh(core_axis_name="c", subcore_axis_name="s")

    @functools.partial(
        pl.kernel, mesh=mesh,
        out_type=jax.ShapeDtypeStruct((b, 2 * DIM), jnp.float32),
        scratch_types=[
            pltpu.VMEM((n_chunk, LANES), jnp.int32),
            pltpu.VMEM((n_buf * LANES, 2 * DIM), jnp.float32),
            pltpu.SemaphoreType.DMA,
        ],
    )
    def gather(table_hbm, pidx_hbm, out_hbm, idx_v, rows_v, sem):
        wid = lax.axis_index("s") * info.num_cores + lax.axis_index("c")
        base = wid * b_per_w
        pltpu.sync_copy(pidx_hbm.at[wid], idx_v)
        for wave in range(n_chunk // n_buf):
            for j in range(n_buf):
                pltpu.async_copy(
                    table_hbm.at[idx_v.at[wave * n_buf + j]],
                    rows_v.at[pl.ds(j * LANES, LANES)], sem)
            for j in range(n_buf):
                pltpu.make_async_copy(
                    table_hbm.at[idx_v.at[wave * n_buf + j]],
                    rows_v.at[pl.ds(j * LANES, LANES)], sem).wait()
            pltpu.sync_copy(
                rows_v,
                out_hbm.at[pl.ds(base + wave * n_buf * LANES, n_buf * LANES)])

    return gather


def _combine_body(w_ref, i_ref, g_ref, out_ref):
    par = (i_ref[...] & 1)[:, :, None] == 1
    rows = jnp.where(par, g_ref[:, :, DIM:], g_ref[:, :, :DIM])
    out_ref[...] = jnp.sum(w_ref[...][:, :, None] * rows, axis=1)


def _run_combine(w, idx, g):
    return pl.pallas_call(
        _combine_body,
        out_shape=jax.ShapeDtypeStruct((Q, DIM), jnp.float32),
    )(w, idx, g)


def kernel(x, queue):
    x2 = x.reshape(Q, DIM)
    qt = queue.T
    w_t, idx_t, pidx_t = _run_topk(qt, x2.T)
    w, idx, pidx = w_t.T, idx_t.T, pidx_t.T
    table = qt.reshape(KDIM // 2, 2 * DIM)
    g = _make_gather()(table, pidx.reshape(32, TOPK * Q // 32 // LANES, LANES))
    out = _run_combine(w, idx, g.reshape(Q, TOPK, 2 * DIM))
    return out.reshape(8, 128, DIM), idx.reshape(8, 128, TOPK)


# speedup vs baseline: 1.3569x; 1.1022x over previous
import functools

import jax
import jax.numpy as jnp
from jax import lax
from jax.experimental import pallas as pl
from jax.experimental.pallas import tpu as pltpu
from jax.experimental.pallas import tpu_sc as plsc

Q = 1024
DIM = 64
KDIM = 65536
TOPK = 32
LANES = 128
FR = 8
TK = 1024
G = TK // FR
NT = KDIM // TK

NEG = float("-inf")


def _lane_top2(simT):
    best = simT[0:FR, :]
    bestg = jnp.zeros((FR, Q), jnp.int32)
    second = jnp.full((FR, Q), NEG)
    secondg = jnp.zeros((FR, Q), jnp.int32)
    for g in range(1, G):
        v = simT[g * FR:(g + 1) * FR, :]
        gt_best = v > best
        gt_second = v > second
        second = jnp.where(gt_best, best, jnp.where(gt_second, v, second))
        secondg = jnp.where(gt_best, bestg,
                            jnp.where(gt_second, g, secondg))
        best = jnp.where(gt_best, v, best)
        bestg = jnp.where(gt_best, g, bestg)
    return best, bestg, second, secondg


def _recompute_second(simT, m):
    second = jnp.full((FR, Q), NEG)
    secondg = jnp.zeros((FR, Q), jnp.int32)
    for g in range(G):
        v = simT[g * FR:(g + 1) * FR, :]
        gt = (v < m) & (v > second)
        secondg = jnp.where(gt, g, secondg)
        second = jnp.where(gt, v, second)
    return second, secondg


def _topk_body(q_ref, x_ref, w_ref, i_ref, p_ref, v_scr, i_scr):
    t = pl.program_id(0)

    @pl.when(t == 0)
    def _():
        v_scr[...] = jnp.full((TOPK, Q), NEG)
        i_scr[...] = jnp.zeros((TOPK, Q), jnp.int32)

    simT = jnp.dot(q_ref[...], x_ref[...], preferred_element_type=jnp.float32)
    m, a, m2, a2 = _lane_top2(simT)

    slot_iota = lax.broadcasted_iota(jnp.int32, (FR, Q), 0)
    is_row0 = lax.broadcasted_iota(jnp.int32, (TOPK, Q), 0) == 0

    def argmax0(arr, mx):
        return jnp.min(jnp.where(arr == mx, slot_iota, FR),
                       axis=0, keepdims=True)

    rv0, ri0 = v_scr[...], i_scr[...]
    mv0 = jnp.max(m, axis=0, keepdims=True)
    lstar0 = argmax0(m, mv0)
    go0 = jnp.any(mv0 > rv0[TOPK - 1:, :])

    def cond(carry):
        return carry[9]

    def body(carry):
        m, a, m2, a2, stale, rv, ri, lstar, mv, go = carry
        active = mv > rv[TOPK - 1:, :]
        ohl = slot_iota == lstar

        stale_at = jnp.max(jnp.where(ohl, stale, 0), axis=0, keepdims=True)
        need = jnp.any((stale_at > 0) & active)
        m2, a2 = lax.cond(need, lambda: _recompute_second(simT, m),
                          lambda: (m2, a2))
        stale = jnp.where(need, jnp.zeros((FR, Q), jnp.int32), stale)

        gstar = jnp.max(jnp.where(ohl, a, 0), axis=0, keepdims=True)
        idx = t * TK + gstar * FR + lstar

        v = jnp.where(active, mv, NEG)
        ge = rv >= v
        sh_v = jnp.concatenate([rv[:1, :], rv[:TOPK - 1, :]], axis=0)
        sh_i = jnp.concatenate([ri[:1, :], ri[:TOPK - 1, :]], axis=0)
        gep = (sh_v >= v) | is_row0
        rv = jnp.where(ge, rv, jnp.where(gep, v, sh_v))
        ri = jnp.where(ge, ri, jnp.where(gep, idx, sh_i))

        onehot = ohl & active
        m = jnp.where(onehot, m2, m)
        a = jnp.where(onehot, a2, a)
        stale = stale + onehot.astype(jnp.int32)

        mv = jnp.max(m, axis=0, keepdims=True)
        lstar = argmax0(m, mv)
        go = jnp.any(mv > rv[TOPK - 1:, :])
        return m, a, m2, a2, stale, rv, ri, lstar, mv, go

    stale0 = jnp.zeros((FR, Q), jnp.int32)
    out = lax.while_loop(cond, body,
                         (m, a, m2, a2, stale0, rv0, ri0, lstar0, mv0, go0))
    v_scr[...] = out[5]
    i_scr[...] = out[6]

    @pl.when(t == NT - 1)
    def _():
        rv = out[5]
        e = jnp.exp(rv - rv[:1, :])
        w_ref[...] = e / jnp.sum(e, axis=0, keepdims=True)
        i_ref[...] = out[6]
        p_ref[...] = lax.shift_right_logical(out[6], 1)


def _run_topk(qt, xT):
    return pl.pallas_call(
        _topk_body,
        grid=(NT,),
        in_specs=[
            pl.BlockSpec((TK, DIM), lambda t: (t, 0)),
            pl.BlockSpec((DIM, Q), lambda t: (0, 0)),
        ],
        out_specs=[
            pl.BlockSpec((TOPK, Q), lambda t: (0, 0)),
            pl.BlockSpec((TOPK, Q), lambda t: (0, 0)),
            pl.BlockSpec((TOPK, Q), lambda t: (0, 0)),
        ],
        out_shape=[
            jax.ShapeDtypeStruct((TOPK, Q), jnp.float32),
            jax.ShapeDtypeStruct((TOPK, Q), jnp.int32),
            jax.ShapeDtypeStruct((TOPK, Q), jnp.int32),
        ],
        scratch_shapes=[
            pltpu.VMEM((TOPK, Q), jnp.float32),
            pltpu.VMEM((TOPK, Q), jnp.int32),
        ],
    )(qt, xT)


@functools.cache
def _make_gather():
    info = plsc.get_sparse_core_info()
    nw = info.num_cores * info.num_subcores
    b = Q * TOPK
    b_per_w = b // nw
    n_chunk = b_per_w // LANES
    n_buf = 4
    mesh = plsc.VectorSubcoreMesh(core_axis_name="c", subcore_axis_name="s")

    @functools.partial(
        pl.kernel, mesh=mesh,
        out_type=jax.ShapeDtypeStruct((b, 2 * DIM), jnp.float32),
        scratch_types=[
            pltpu.VMEM((n_chunk, LANES), jnp.int32),
            pltpu.VMEM((n_buf * LANES, 2 * DIM), jnp.float32),
            pltpu.SemaphoreType.DMA,
        ],
    )
    def gather(table_hbm, pidx_hbm, out_hbm, idx_v, rows_v, sem):
        wid = lax.axis_index("s") * info.num_cores + lax.axis_index("c")
        base = wid * b_per_w
        pltpu.sync_copy(pidx_hbm.at[wid], idx_v)
        for wave in range(n_chunk // n_buf):
            for j in range(n_buf):
                pltpu.async_copy(
                    table_hbm.at[idx_v.at[wave * n_buf + j]],
                    rows_v.at[pl.ds(j * LANES, LANES)], sem)
            for j in range(n_buf):
                pltpu.make_async_copy(
                    table_hbm.at[idx_v.at[wave * n_buf + j]],
                    rows_v.at[pl.ds(j * LANES, LANES)], sem).wait()
            pltpu.sync_copy(
                rows_v,
                out_hbm.at[pl.ds(base + wave * n_buf * LANES, n_buf * LANES)])

    return gather


def _combine_body(w_ref, i_ref, g_ref, out_ref):
    par = (i_ref[...] & 1)[:, :, None] == 1
    rows = jnp.where(par, g_ref[:, :, DIM:], g_ref[:, :, :DIM])
    out_ref[...] = jnp.sum(w_ref[...][:, :, None] * rows, axis=1)


def _run_combine(w, idx, g):
    return pl.pallas_call(
        _combine_body,
        out_shape=jax.ShapeDtypeStruct((Q, DIM), jnp.float32),
    )(w, idx, g)


def kernel(x, queue):
    x2 = x.reshape(Q, DIM)
    qt = queue.T
    w_t, idx_t, pidx_t = _run_topk(qt, x2.T)
    w, idx, pidx = w_t.T, idx_t.T, pidx_t.T
    table = qt.reshape(KDIM // 2, 2 * DIM)
    g = _make_gather()(table, pidx.reshape(32, TOPK * Q // 32 // LANES, LANES))
    out = _run_combine(w, idx, g.reshape(Q, TOPK, 2 * DIM))
    return out.reshape(8, 128, DIM), idx.reshape(8, 128, TOPK)
